# Initial kernel scaffold; baseline (speedup 1.0000x reference)
import jax
import jax.numpy as jnp
from jax.experimental import pallas as pl


def kernel(x, edge_index, batch, W1, b1, W2, b2, Wlin, blin):
    raise NotImplementedError("write your pallas kernel here")


# speedup vs baseline: 21.8596x; 21.8596x over previous
import jax
import jax.numpy as jnp
from jax import lax
from jax.experimental import pallas as pl
from jax.experimental.pallas import tpu as pltpu
from jax.experimental.pallas import tpu_sc as plsc

N = 10000
E = 320000
D = 128
C = 10
NC, NS = 2, 16
NW = NC * NS
EPW = E // NW
K = 125
NCH = EPW // K
NP = 10240
RPS = NP // NS
HR = NP // D
EV = EPW // 16
R = 1000
GRID = N // R

_MESH = plsc.VectorSubcoreMesh(core_axis_name="c", subcore_axis_name="s")


def _deg_body(dst_hbm, zz_hbm, out_hbm, dst_v, hist_v):
    c = lax.axis_index("c")
    s = lax.axis_index("s")
    w = c * NS + s
    pltpu.sync_copy(zz_hbm, hist_v)
    pltpu.sync_copy(dst_hbm.at[w], dst_v)
    ones16 = jnp.full((16,), 1.0, jnp.float32)
    lane = lax.iota(jnp.int32, 16)

    def body(i, carry):
        idx = plsc.load_gather(dst_v, [i * 16 + lane])
        plsc.addupdate_scatter(hist_v, [lax.shift_right_logical(idx, 7),
                                        lax.bitwise_and(idx, 127)], ones16)
        return carry

    lax.fori_loop(0, EV, body, 0)
    pltpu.sync_copy(hist_v, out_hbm.at[w])


_deg = pl.kernel(
    _deg_body,
    out_type=jax.ShapeDtypeStruct((NW, HR, D), jnp.float32),
    mesh=_MESH,
    compiler_params=pltpu.CompilerParams(needs_layout_passes=False),
    scratch_types=[
        pltpu.VMEM((EPW,), jnp.int32),
        pltpu.VMEM((HR, D), jnp.float32),
    ],
)


def _agg_body(xs_hbm, src_hbm, dst_hbm, zz_hbm, out_hbm,
              src_v, dst_v, rows_v, acc_sh, sem):
    c = lax.axis_index("c")
    s = lax.axis_index("s")
    w = c * NS + s
    pltpu.sync_copy(zz_hbm, acc_sh.at[pl.ds(s * RPS, RPS)])
    pltpu.sync_copy(src_hbm.at[w], src_v)
    pltpu.sync_copy(dst_hbm.at[w], dst_v)
    plsc.subcore_barrier()

    def body(j, carry):
        pltpu.async_copy(xs_hbm.at[src_v.at[j]], rows_v, sem).wait()
        pltpu.sync_copy(rows_v, acc_sh.at[dst_v.at[j]], add=True)
        return carry

    lax.fori_loop(0, NCH, body, 0)
    plsc.subcore_barrier()
    pltpu.sync_copy(acc_sh.at[pl.ds(s * RPS, RPS)],
                    out_hbm.at[c, pl.ds(s * RPS, RPS)])


_agg = pl.kernel(
    _agg_body,
    out_type=jax.ShapeDtypeStruct((NC, NP, D), jnp.float32),
    mesh=_MESH,
    scratch_types=[
        pltpu.VMEM((NCH, K), jnp.int32),
        pltpu.VMEM((NCH, K), jnp.int32),
        pltpu.VMEM((K, D), jnp.float32),
        pltpu.VMEM_SHARED((NP, D), jnp.float32),
        pltpu.SemaphoreType.DMA,
    ],
)


def _dinv_body(h_ref, o_ref):
    deg = jnp.sum(h_ref[...], axis=0) + 1.0
    o_ref[...] = lax.rsqrt(deg)


_dinvk = pl.pallas_call(
    _dinv_body,
    grid=(1,),
    in_specs=[pl.BlockSpec((NW, HR, D), lambda i: (0, 0, 0))],
    out_specs=pl.BlockSpec((HR, D), lambda i: (0, 0)),
    out_shape=jax.ShapeDtypeStruct((HR, D), jnp.float32),
)


def _mm_scale_body(x_ref, w_ref, dv_ref, o_ref):
    xw = jnp.dot(x_ref[...], w_ref[...], preferred_element_type=jnp.float32,
                 precision=lax.Precision.HIGHEST)
    o_ref[...] = xw * dv_ref[...]


def _layer2_body(s0_ref, s1_ref, xs_ref, dv_ref, b_ref, w_ref, o_ref):
    dinv = dv_ref[...]
    h = (s0_ref[...] + s1_ref[...] + xs_ref[...]) * dinv + b_ref[...]
    h = jnp.maximum(h, 0.0)
    o_ref[...] = jnp.dot(h, w_ref[...], preferred_element_type=jnp.float32,
                         precision=lax.Precision.HIGHEST) * dinv


def _final_body(t0_ref, t1_ref, xs_ref, dv_ref, b_ref, wl_ref, bl_ref, o_ref):
    h = (t0_ref[...] + t1_ref[...] + xs_ref[...]) * dv_ref[...] + b_ref[...]
    o_ref[...] = jnp.dot(h, wl_ref[...], preferred_element_type=jnp.float32,
                         precision=lax.Precision.HIGHEST) + bl_ref[...]


def _row_spec(cols):
    return pl.BlockSpec((R, cols), lambda i: (i, 0))


def _full_spec(rows, cols):
    return pl.BlockSpec((rows, cols), lambda i: (0, 0))


_mm_scale = pl.pallas_call(
    _mm_scale_body,
    grid=(GRID,),
    in_specs=[_row_spec(D), _full_spec(D, D), _row_spec(1)],
    out_specs=_row_spec(D),
    out_shape=jax.ShapeDtypeStruct((N, D), jnp.float32),
)

_layer2 = pl.pallas_call(
    _layer2_body,
    grid=(GRID,),
    in_specs=[_row_spec(D), _row_spec(D), _row_spec(D), _row_spec(1),
              _full_spec(1, D), _full_spec(D, D)],
    out_specs=_row_spec(D),
    out_shape=jax.ShapeDtypeStruct((N, D), jnp.float32),
)

_final = pl.pallas_call(
    _final_body,
    grid=(GRID,),
    in_specs=[_row_spec(D), _row_spec(D), _row_spec(D), _row_spec(1),
              _full_spec(1, D), _full_spec(D, C), _full_spec(1, C)],
    out_specs=_row_spec(C),
    out_shape=jax.ShapeDtypeStruct((N, C), jnp.float32),
)


def kernel(x, edge_index, batch, W1, b1, W2, b2, Wlin, blin):
    src = edge_index[0].astype(jnp.int32)
    dst = edge_index[1].astype(jnp.int32)
    src3 = src.reshape(NW, NCH, K)
    dst3 = dst.reshape(NW, NCH, K)
    dst2 = dst.reshape(NW, EPW)
    zz_d = jnp.zeros((RPS, D), jnp.float32)
    zz_h = jnp.zeros((HR, D), jnp.float32)

    hists = _deg(dst2, zz_h)
    dv = _dinvk(hists).reshape(NP, 1)[:N]

    xs1 = _mm_scale(x, W1, dv)
    s_part = _agg(xs1, src3, dst3, zz_d)
    xs2 = _layer2(s_part[0, :N], s_part[1, :N], xs1, dv, b1.reshape(1, D), W2)
    t_part = _agg(xs2, src3, dst3, zz_d)
    return _final(t_part[0, :N], t_part[1, :N], xs2, dv,
                  b2.reshape(1, D), Wlin, blin.reshape(1, C))


# speedup vs baseline: 28.5960x; 1.3082x over previous
import jax
import jax.numpy as jnp
from jax import lax
from jax.experimental import pallas as pl
from jax.experimental.pallas import tpu as pltpu
from jax.experimental.pallas import tpu_sc as plsc

N = 10000
E = 320000
D = 128
C = 10
NC, NS = 2, 16
NW = NC * NS
EPW = E // NW
K = 125
NCH = EPW // K
NP = 10240
RPS = NP // NS
HR = NP // D
EV = EPW // 16
R = 1000
GRID = N // R

_MESH = plsc.VectorSubcoreMesh(core_axis_name="c", subcore_axis_name="s")


def _deg_body(dst_hbm, zz_hbm, out_hbm, dst_v, hist_v):
    c = lax.axis_index("c")
    s = lax.axis_index("s")
    w = c * NS + s
    pltpu.sync_copy(zz_hbm, hist_v)
    pltpu.sync_copy(dst_hbm.at[w], dst_v)
    ones16 = jnp.full((16,), 1.0, jnp.float32)
    lane = lax.iota(jnp.int32, 16)

    def body(i, carry):
        idx = plsc.load_gather(dst_v, [i * 16 + lane])
        plsc.addupdate_scatter(hist_v, [lax.shift_right_logical(idx, 7),
                                        lax.bitwise_and(idx, 127)], ones16)
        return carry

    lax.fori_loop(0, EV, body, 0)
    pltpu.sync_copy(hist_v, out_hbm.at[w])


_deg = pl.kernel(
    _deg_body,
    out_type=jax.ShapeDtypeStruct((NW, HR, D), jnp.float32),
    mesh=_MESH,
    compiler_params=pltpu.CompilerParams(needs_layout_passes=False),
    scratch_types=[
        pltpu.VMEM((EPW,), jnp.int32),
        pltpu.VMEM((HR, D), jnp.float32),
    ],
)


def _agg_body(xs_hbm, src_hbm, dst_hbm, zz_hbm, out_hbm,
              ibuf_v, dst_v, rows_v, acc_sh, sem_i, sem_g, sem_z):
    c = lax.axis_index("c")
    s = lax.axis_index("s")
    w = c * NS + s
    zdesc = pltpu.async_copy(zz_hbm, acc_sh.at[pl.ds(s * RPS, RPS)], sem_z)
    pltpu.sync_copy(dst_hbm.at[w], dst_v)
    pltpu.sync_copy(src_hbm.at[w, 0], ibuf_v.at[0])
    pltpu.async_copy(src_hbm.at[w, 1], ibuf_v.at[1], sem_i.at[1])
    zdesc.wait()
    plsc.subcore_barrier()

    pltpu.async_copy(xs_hbm.at[ibuf_v.at[0]], rows_v.at[0], sem_g.at[0])

    def body(j, carry):
        cur = lax.rem(j, 2)
        nxt = lax.rem(j + 1, 2)
        i1 = lax.rem(j + 1, 3)
        i2 = lax.rem(j + 2, 3)

        @pl.when(j + 2 < NCH)
        def _():
            pltpu.async_copy(src_hbm.at[w, j + 2], ibuf_v.at[i2],
                             sem_i.at[i2])

        pltpu.make_async_copy(xs_hbm.at[ibuf_v.at[lax.rem(j, 3)]],
                              rows_v.at[cur], sem_g.at[cur]).wait()

        @pl.when(j + 1 < NCH)
        def _():
            pltpu.make_async_copy(src_hbm.at[w, j + 1], ibuf_v.at[i1],
                                  sem_i.at[i1]).wait()
            pltpu.async_copy(xs_hbm.at[ibuf_v.at[i1]], rows_v.at[nxt],
                             sem_g.at[nxt])

        pltpu.sync_copy(rows_v.at[cur], acc_sh.at[dst_v.at[j]], add=True)
        return carry

    lax.fori_loop(0, NCH, body, 0)
    plsc.subcore_barrier()
    pltpu.sync_copy(acc_sh.at[pl.ds(s * RPS, RPS)],
                    out_hbm.at[c, pl.ds(s * RPS, RPS)])


_agg = pl.kernel(
    _agg_body,
    out_type=jax.ShapeDtypeStruct((NC, NP, D), jnp.float32),
    mesh=_MESH,
    scratch_types=[
        pltpu.VMEM((3, K), jnp.int32),
        pltpu.VMEM((NCH, K), jnp.int32),
        pltpu.VMEM((2, K, D), jnp.float32),
        pltpu.VMEM_SHARED((NP, D), jnp.float32),
        pltpu.SemaphoreType.DMA((3,)),
        pltpu.SemaphoreType.DMA((2,)),
        pltpu.SemaphoreType.DMA,
    ],
)


def _dinv_body(h_ref, o_ref):
    deg = jnp.sum(h_ref[...], axis=0) + 1.0
    o_ref[...] = lax.rsqrt(deg)


_dinvk = pl.pallas_call(
    _dinv_body,
    grid=(1,),
    in_specs=[pl.BlockSpec((NW, HR, D), lambda i: (0, 0, 0))],
    out_specs=pl.BlockSpec((HR, D), lambda i: (0, 0)),
    out_shape=jax.ShapeDtypeStruct((HR, D), jnp.float32),
)


def _mm_scale_body(x_ref, w_ref, dv_ref, o_ref):
    xw = jnp.dot(x_ref[...], w_ref[...], preferred_element_type=jnp.float32,
                 precision=lax.Precision.HIGHEST)
    o_ref[...] = xw * dv_ref[...]


def _layer2_body(s0_ref, s1_ref, xs_ref, dv_ref, b_ref, w_ref, o_ref):
    dinv = dv_ref[...]
    h = (s0_ref[0] + s1_ref[0] + xs_ref[...]) * dinv + b_ref[...]
    h = jnp.maximum(h, 0.0)
    o_ref[...] = jnp.dot(h, w_ref[...], preferred_element_type=jnp.float32,
                         precision=lax.Precision.HIGHEST) * dinv


def _final_body(t0_ref, t1_ref, xs_ref, dv_ref, b_ref, wl_ref, bl_ref, o_ref):
    h = (t0_ref[0] + t1_ref[0] + xs_ref[...]) * dv_ref[...] + b_ref[...]
    o_ref[...] = jnp.dot(h, wl_ref[...], preferred_element_type=jnp.float32,
                         precision=lax.Precision.HIGHEST) + bl_ref[...]


def _row_spec(cols):
    return pl.BlockSpec((R, cols), lambda i: (i, 0))


def _part_spec(core):
    return pl.BlockSpec((1, R, D), lambda i, core=core: (core, i, 0))


def _full_spec(rows, cols):
    return pl.BlockSpec((rows, cols), lambda i: (0, 0))


_mm_scale = pl.pallas_call(
    _mm_scale_body,
    grid=(GRID,),
    in_specs=[_row_spec(D), _full_spec(D, D), _row_spec(1)],
    out_specs=_row_spec(D),
    out_shape=jax.ShapeDtypeStruct((N, D), jnp.float32),
)

_layer2 = pl.pallas_call(
    _layer2_body,
    grid=(GRID,),
    in_specs=[_part_spec(0), _part_spec(1), _row_spec(D), _row_spec(1),
              _full_spec(1, D), _full_spec(D, D)],
    out_specs=_row_spec(D),
    out_shape=jax.ShapeDtypeStruct((N, D), jnp.float32),
)

_final = pl.pallas_call(
    _final_body,
    grid=(GRID,),
    in_specs=[_part_spec(0), _part_spec(1), _row_spec(D), _row_spec(1),
              _full_spec(1, D), _full_spec(D, C), _full_spec(1, C)],
    out_specs=_row_spec(C),
    out_shape=jax.ShapeDtypeStruct((N, C), jnp.float32),
)


def kernel(x, edge_index, batch, W1, b1, W2, b2, Wlin, blin):
    src = edge_index[0].astype(jnp.int32)
    dst = edge_index[1].astype(jnp.int32)
    src3 = src.reshape(NW, NCH, K)
    dst3 = dst.reshape(NW, NCH, K)
    dst2 = dst.reshape(NW, EPW)
    zz_d = jnp.zeros((RPS, D), jnp.float32)
    zz_h = jnp.zeros((HR, D), jnp.float32)

    hists = _deg(dst2, zz_h)
    dv = _dinvk(hists).reshape(NP, 1)

    xs1 = _mm_scale(x, W1, dv)
    s_part = _agg(xs1, src3, dst3, zz_d)
    xs2 = _layer2(s_part, s_part, xs1, dv, b1.reshape(1, D), W2)
    t_part = _agg(xs2, src3, dst3, zz_d)
    return _final(t_part, t_part, xs2, dv,
                  b2.reshape(1, D), Wlin, blin.reshape(1, C))
